# baseline (speedup 1.0000x reference)
import jax
import jax.numpy as jnp
from jax.experimental import pallas as pl
from jax.experimental.pallas import tpu as pltpu

_CHUNK = 200
_NBUF = 4


def _postproc(s1, b1):
    z = s1 + b1
    m = jnp.max(z, axis=1, keepdims=True)
    s = z - m
    return s - jnp.log(jnp.sum(jnp.exp(s), axis=1, keepdims=True))


def _gcn_body(x_ref, w0_ref, b0_ref, w1_ref, b1_ref, adj_hbm, o_ref,
              s0_ref, buf0, buf1, buf2, buf3, sem0, sem1, sem2, sem3):
    n = x_ref.shape[0]
    nchunks = n // _CHUNK
    bufs = (buf0, buf1, buf2, buf3)
    sems = (sem0, sem1, sem2, sem3)

    for c in range(_NBUF - 1):
        pltpu.make_async_copy(
            adj_hbm.at[pl.ds(c * _CHUNK, _CHUNK)], bufs[c], sems[c]).start()
    s0_ref[...] = jnp.dot(x_ref[...], w0_ref[...],
                          preferred_element_type=jnp.float32)

    def step(i, carry):
        del carry
        for parity in range(_NBUF):
            @pl.when((i % _NBUF) == parity)
            def _():
                buf, sem = bufs[parity], sems[parity]
                np_ = (parity + _NBUF - 1) % _NBUF
                nbuf, nsem = bufs[np_], sems[np_]

                @pl.when(i + _NBUF - 1 < nchunks)
                def _():
                    pltpu.make_async_copy(
                        adj_hbm.at[pl.ds((i + _NBUF - 1) * _CHUNK, _CHUNK)],
                        nbuf, nsem).start()

                pltpu.make_async_copy(
                    adj_hbm.at[pl.ds(i * _CHUNK, _CHUNK)], buf, sem).wait()
                h = jnp.dot(buf[...], s0_ref[...],
                            preferred_element_type=jnp.float32)
                h = jnp.maximum(h + b0_ref[...], 0.0)
                s1 = jnp.dot(h, w1_ref[...],
                             preferred_element_type=jnp.float32)
                o_ref[pl.ds(i * _CHUNK, _CHUNK), :] = _postproc(
                    s1, b1_ref[...])
        return 0

    jax.lax.fori_loop(0, nchunks, step, 0)


def kernel(x, adj, W0, b0, W1, b1):
    n, nfeat = x.shape
    nhid = W0.shape[1]
    nclass = W1.shape[1]

    out = pl.pallas_call(
        _gcn_body,
        in_specs=[
            pl.BlockSpec(memory_space=pltpu.VMEM),
            pl.BlockSpec(memory_space=pltpu.VMEM),
            pl.BlockSpec(memory_space=pltpu.VMEM),
            pl.BlockSpec(memory_space=pltpu.VMEM),
            pl.BlockSpec(memory_space=pltpu.VMEM),
            pl.BlockSpec(memory_space=pl.ANY),
        ],
        out_specs=pl.BlockSpec(memory_space=pltpu.VMEM),
        out_shape=jax.ShapeDtypeStruct((n, nclass), jnp.float32),
        scratch_shapes=[
            pltpu.VMEM((n, nhid), jnp.float32),
            pltpu.VMEM((_CHUNK, n), jnp.float32),
            pltpu.VMEM((_CHUNK, n), jnp.float32),
            pltpu.VMEM((_CHUNK, n), jnp.float32),
            pltpu.VMEM((_CHUNK, n), jnp.float32),
            pltpu.SemaphoreType.DMA,
            pltpu.SemaphoreType.DMA,
            pltpu.SemaphoreType.DMA,
            pltpu.SemaphoreType.DMA,
        ],
        compiler_params=pltpu.CompilerParams(
            vmem_limit_bytes=120 * 1024 * 1024,
        ),
    )(x, W0, b0.reshape(1, nhid), W1, b1.reshape(1, nclass), adj)

    return out
